# baseline (speedup 1.0000x reference)
import functools

import jax
import jax.numpy as jnp
from jax import lax
from jax.experimental import pallas as pl
from jax.experimental.pallas import tpu as pltpu
from jax.experimental.pallas import tpu_sc as plsc

D = 768
NTOK = 2048
BATCH = 4
B = BATCH * NTOK
L = 16

NC = 2
NS = 16
NW = NC * NS
BPW = B // NW
C = 32
NCHUNK = BPW // C
CG = D // L

_mesh = plsc.VectorSubcoreMesh(core_axis_name="c", subcore_axis_name="s")


@functools.partial(
    pl.kernel,
    mesh=_mesh,
    out_type=jax.ShapeDtypeStruct((B, D), jnp.float32),
    scratch_types=[
        pltpu.VMEM((BPW,), jnp.int32),
        pltpu.VMEM((C, D), jnp.float32),
        pltpu.VMEM((C, D), jnp.float32),
        pltpu.SemaphoreType.DMA,
        pltpu.SemaphoreType.DMA,
    ],
)
def _emb_kernel(idx_hbm, table_hbm, pe_hbm, out_hbm, idx_v, gbuf, pbuf, gsem, psem):
    wid = lax.axis_index("s") * NC + lax.axis_index("c")
    base = wid * BPW
    pos0 = lax.rem(base, NTOK)
    pltpu.sync_copy(idx_hbm.at[pl.ds(base, BPW)], idx_v)
    for ci in range(NCHUNK):
        g = pltpu.async_copy(table_hbm.at[idx_v.at[pl.ds(ci * C, C)]], gbuf, gsem)
        p = pltpu.async_copy(pe_hbm.at[pl.ds(pos0 + ci * C, C), :], pbuf, psem)
        g.wait()
        p.wait()

        def row_body(r, carry):
            for cg in range(CG):
                s = pl.ds(cg * L, L)
                gbuf[r, s] = gbuf[r, s] + pbuf[r, s]
            return carry

        lax.fori_loop(0, C, row_body, 0)
        pltpu.sync_copy(gbuf, out_hbm.at[pl.ds(base + ci * C, C), :])


def kernel(x, embed_weight, positional_encoding):
    idx = x.reshape(-1).astype(jnp.int32)
    out = _emb_kernel(idx, embed_weight, positional_encoding)
    return out.reshape(x.shape[0], x.shape[1], D)


# speedup vs baseline: 1.1115x; 1.1115x over previous
import functools

import jax
import jax.numpy as jnp
from jax import lax
from jax.experimental import pallas as pl
from jax.experimental.pallas import tpu as pltpu
from jax.experimental.pallas import tpu_sc as plsc

D = 768
NTOK = 2048
BATCH = 4
B = BATCH * NTOK
L = 16

NC = 2
NS = 16
NW = NC * NS
PPW = NTOK // NW
C = 32
NCHUNK = BATCH * PPW // C
CG = D // L
NBUF = 3

_mesh = plsc.VectorSubcoreMesh(core_axis_name="c", subcore_axis_name="s")


@functools.partial(
    pl.kernel,
    mesh=_mesh,
    out_type=jax.ShapeDtypeStruct((B, D), jnp.float32),
    scratch_types=[
        pltpu.VMEM((BATCH * PPW,), jnp.int32),
        pltpu.VMEM((PPW, D), jnp.float32),
        pltpu.VMEM((C, D), jnp.float32),
        pltpu.VMEM((C, D), jnp.float32),
        pltpu.VMEM((C, D), jnp.float32),
        pltpu.SemaphoreType.DMA,
        pltpu.SemaphoreType.DMA,
        pltpu.SemaphoreType.DMA,
    ],
)
def _emb_kernel(idx_hbm, table_hbm, pe_hbm, out_hbm,
                idx_v, pbuf, b0, b1, b2, gsem, psem, osem):
    wid = lax.axis_index("s") * NC + lax.axis_index("c")
    p0 = wid * PPW
    for b in range(BATCH):
        pltpu.sync_copy(idx_hbm.at[pl.ds(b * NTOK + p0, PPW)],
                        idx_v.at[pl.ds(b * PPW, PPW)])
    bufs = [b0, b1, b2]
    gh = [pltpu.async_copy(table_hbm.at[idx_v.at[pl.ds(0, C)]], bufs[0], gsem)]
    ph = pltpu.async_copy(pe_hbm.at[pl.ds(p0, PPW), :], pbuf, psem)
    ph.wait()
    oh = []
    for ci in range(NCHUNK):
        if ci + 1 < NCHUNK:
            if ci + 1 >= NBUF:
                oh[ci + 1 - NBUF].wait()
            gh.append(pltpu.async_copy(
                table_hbm.at[idx_v.at[pl.ds((ci + 1) * C, C)]],
                bufs[(ci + 1) % NBUF], gsem))
        gh[ci].wait()
        buf = bufs[ci % NBUF]
        h = ci % 2

        def row_body(r, carry):
            for cg in range(CG):
                s = pl.ds(cg * L, L)
                buf[r, s] = buf[r, s] + pbuf[h * C + r, s]
            return carry

        lax.fori_loop(0, C, row_body, 0)
        b = ci // 2
        oh.append(pltpu.async_copy(
            buf, out_hbm.at[pl.ds(b * NTOK + p0 + h * C, C), :], osem))
    for hh in oh[NCHUNK - NBUF + 1:]:
        hh.wait()


def kernel(x, embed_weight, positional_encoding):
    idx = x.reshape(-1).astype(jnp.int32)
    out = _emb_kernel(idx, embed_weight, positional_encoding)
    return out.reshape(x.shape[0], x.shape[1], D)
